# Initial kernel scaffold; baseline (speedup 1.0000x reference)
import jax
import jax.numpy as jnp
from jax.experimental import pallas as pl


def kernel(pieces_ids, color_ids, position_table, piece_table, color_table, indexes):
    raise NotImplementedError("write your pallas kernel here")


# speedup vs baseline: 7.1380x; 7.1380x over previous
import functools

import jax
import jax.numpy as jnp
from jax import lax
from jax.experimental import pallas as pl
from jax.experimental.pallas import tpu as pltpu
from jax.experimental.pallas import tpu_sc as plsc

D = 64
C = 512
G = 128


def _sc_embed(fused_table, pieces_flat, colors_flat):
    N = pieces_flat.shape[1]
    mesh = plsc.VectorSubcoreMesh(core_axis_name="core",
                                  subcore_axis_name="subcore")

    @functools.partial(
        pl.kernel,
        out_type=jax.ShapeDtypeStruct((N, D), jnp.float32),
        mesh=mesh,
        scratch_types=[pltpu.VMEM((C // G, G), jnp.int32)],
        compiler_params=pltpu.CompilerParams(use_tc_tiling_on_sc=False),
    )
    def k(table_hbm, p_hbm, c_hbm, out_hbm, idx_v):
        def body(p_vmem, c_vmem, out_vmem):
            iota = lax.iota(jnp.int32, 16)

            @pl.loop(0, C, step=16)
            def _(i):
                p = p_vmem[0, pl.ds(i, 16)]
                c = c_vmem[0, pl.ds(i, 16)]
                s = iota + (i % 64)
                idx_v[i // G, pl.ds(i % G, 16)] = p * 192 + c * 64 + s

            @pl.loop(0, C // G)
            def _(j):
                pltpu.sync_copy(table_hbm.at[idx_v.at[j]],
                                out_vmem.at[pl.ds(j * G, G), :])

        pltpu.emit_pipeline(
            body,
            grid=(N // C,),
            in_specs=[pl.BlockSpec((1, C), lambda i: (0, i)),
                      pl.BlockSpec((1, C), lambda i: (0, i))],
            out_specs=[pl.BlockSpec((C, D), lambda i: (i, 0))],
            core_axis_name=("core", "subcore"),
            dimension_semantics=(pltpu.PARALLEL,),
        )(p_hbm, c_hbm, out_hbm)

    return k(fused_table, pieces_flat, colors_flat)


def kernel(pieces_ids, color_ids, position_table, piece_table, color_table,
           indexes):
    B, S = pieces_ids.shape
    pos = jnp.take(position_table, indexes, axis=0)
    fused = (piece_table[:, None, None, :]
             + color_table[None, :, None, :]
             + pos[None, None, :, :]).reshape(-1, D)
    p = pieces_ids.reshape(1, B * S).astype(jnp.int32)
    c = color_ids.reshape(1, B * S).astype(jnp.int32)
    out = _sc_embed(fused, p, c)
    return out.reshape(B, S, D)


# speedup vs baseline: 7.5050x; 1.0514x over previous
import functools

import jax
import jax.numpy as jnp
from jax import lax
from jax.experimental import pallas as pl
from jax.experimental.pallas import tpu as pltpu
from jax.experimental.pallas import tpu_sc as plsc

D = 64
C = 512
G = 128


def _sc_embed(fused_table, pieces_flat, colors_flat):
    N = pieces_flat.shape[1]
    mesh = plsc.VectorSubcoreMesh(core_axis_name="core",
                                  subcore_axis_name="subcore")

    @functools.partial(
        pl.kernel,
        out_type=jax.ShapeDtypeStruct((N, D), jnp.float32),
        mesh=mesh,
        scratch_types=[pltpu.VMEM((C // G, G), jnp.int32),
                       pltpu.SemaphoreType.DMA],
        compiler_params=pltpu.CompilerParams(use_tc_tiling_on_sc=False),
    )
    def k(table_hbm, p_hbm, c_hbm, out_hbm, idx_v, sem):
        def body(p_vmem, c_vmem, out_vmem):
            iota = lax.iota(jnp.int32, 16)

            @pl.loop(0, C, step=16)
            def _(i):
                p = p_vmem[0, pl.ds(i, 16)]
                c = c_vmem[0, pl.ds(i, 16)]
                s = iota + (i % 64)
                idx_v[i // G, pl.ds(i % G, 16)] = p * 192 + c * 64 + s

            copies = [
                pltpu.async_copy(table_hbm.at[idx_v.at[j]],
                                 out_vmem.at[pl.ds(j * G, G), :], sem)
                for j in range(C // G)
            ]
            for cp in copies:
                cp.wait()

        pltpu.emit_pipeline(
            body,
            grid=(N // C,),
            in_specs=[pl.BlockSpec((1, C), lambda i: (0, i)),
                      pl.BlockSpec((1, C), lambda i: (0, i))],
            out_specs=[pl.BlockSpec((C, D), lambda i: (i, 0))],
            core_axis_name=("core", "subcore"),
            dimension_semantics=(pltpu.PARALLEL,),
        )(p_hbm, c_hbm, out_hbm)

    return k(fused_table, pieces_flat, colors_flat)


def kernel(pieces_ids, color_ids, position_table, piece_table, color_table,
           indexes):
    B, S = pieces_ids.shape
    pos = jnp.take(position_table, indexes, axis=0)
    fused = (piece_table[:, None, None, :]
             + color_table[None, :, None, :]
             + pos[None, None, :, :]).reshape(-1, D)
    p = pieces_ids.reshape(1, B * S).astype(jnp.int32)
    c = color_ids.reshape(1, B * S).astype(jnp.int32)
    out = _sc_embed(fused, p, c)
    return out.reshape(B, S, D)


# speedup vs baseline: 10.8975x; 1.4520x over previous
import functools

import jax
import jax.numpy as jnp
from jax import lax
from jax.experimental import pallas as pl
from jax.experimental.pallas import tpu as pltpu
from jax.experimental.pallas import tpu_sc as plsc

D = 64
S = 64
BBLK = 256
BLKB = 1024


def _sc_pc_transpose(pieces_flat, colors_flat, B):
    mesh = plsc.VectorSubcoreMesh(core_axis_name="core",
                                  subcore_axis_name="subcore")

    @functools.partial(
        pl.kernel,
        out_type=jax.ShapeDtypeStruct((S, B), jnp.int32),
        mesh=mesh,
        scratch_types=[pltpu.VMEM((BBLK * S,), jnp.int32)],
        compiler_params=pltpu.CompilerParams(use_tc_tiling_on_sc=False,
                                             needs_layout_passes=False),
    )
    def k(p_hbm, c_hbm, out_hbm, pc_v):
        def body(p_vmem, c_vmem, out_vmem):
            @pl.loop(0, BBLK * S, step=16)
            def _(i):
                pc_v[pl.ds(i, 16)] = (p_vmem[0, pl.ds(i, 16)] * 3
                                      + c_vmem[0, pl.ds(i, 16)])

            iota64 = lax.iota(jnp.int32, 16) * S

            @pl.loop(0, S)
            def _(s):
                @pl.loop(0, BBLK, step=16)
                def _(j):
                    idx = iota64 + (j * S + s)
                    out_vmem[s, pl.ds(j, 16)] = plsc.load_gather(pc_v, [idx])

        pltpu.emit_pipeline(
            body,
            grid=(B // BBLK,),
            in_specs=[pl.BlockSpec((1, BBLK * S), lambda i: (0, i)),
                      pl.BlockSpec((1, BBLK * S), lambda i: (0, i))],
            out_specs=[pl.BlockSpec((S, BBLK), lambda i: (0, i))],
            core_axis_name=("core", "subcore"),
            dimension_semantics=(pltpu.PARALLEL,),
        )(p_hbm, c_hbm, out_hbm)

    return k(pieces_flat, colors_flat)


def _tc_body(tbl_ref, pc_ref, out_ref):
    tbl = tbl_ref[0]
    idx = pc_ref[0]
    idxb = jnp.broadcast_to(idx, (D, BLKB))
    out_ref[0] = jnp.take_along_axis(tbl, idxb, axis=1)


def _tc_lookup(tbl, pcT):
    B = pcT.shape[2]
    return pl.pallas_call(
        _tc_body,
        grid=(S, B // BLKB),
        in_specs=[pl.BlockSpec((1, D, 128), lambda s, j: (s, 0, 0)),
                  pl.BlockSpec((1, 1, BLKB), lambda s, j: (s, 0, j))],
        out_specs=pl.BlockSpec((1, D, BLKB), lambda s, j: (s, 0, j)),
        out_shape=jax.ShapeDtypeStruct((S, D, B), jnp.float32),
    )(tbl, pcT)


def kernel(pieces_ids, color_ids, position_table, piece_table, color_table,
           indexes):
    B, _ = pieces_ids.shape
    pos = jnp.take(position_table, indexes, axis=0)
    pcv = jnp.arange(128)
    pieceT = piece_table[jnp.clip(pcv // 3, 0, 6)].T
    colorT = color_table[pcv % 3].T
    tbl = pos[:, :, None] + (pieceT + colorT)[None, :, :]
    p = pieces_ids.reshape(1, B * S).astype(jnp.int32)
    c = color_ids.reshape(1, B * S).astype(jnp.int32)
    pcT = _sc_pc_transpose(p, c, B)
    out3 = _tc_lookup(tbl, pcT.reshape(S, 1, B))
    return jnp.transpose(out3, (2, 0, 1))


# speedup vs baseline: 17.1282x; 1.5718x over previous
import functools

import jax
import jax.numpy as jnp
from jax import lax
from jax.experimental import pallas as pl
from jax.experimental.pallas import tpu as pltpu
from jax.experimental.pallas import tpu_sc as plsc

D = 64
S = 64
BBLK = 256
BLKB = 2048


def _sc_pc_transpose(pieces_flat, colors_flat, B):
    mesh = plsc.VectorSubcoreMesh(core_axis_name="core",
                                  subcore_axis_name="subcore")

    @functools.partial(
        pl.kernel,
        out_type=jax.ShapeDtypeStruct((S, B), jnp.int32),
        mesh=mesh,
        scratch_types=[pltpu.VMEM((BBLK * S,), jnp.int32)],
        compiler_params=pltpu.CompilerParams(use_tc_tiling_on_sc=False,
                                             needs_layout_passes=False),
    )
    def k(p_hbm, c_hbm, out_hbm, pc_v):
        def body(p_vmem, c_vmem, out_vmem):
            @pl.loop(0, BBLK * S, step=16)
            def _(i):
                pc_v[pl.ds(i, 16)] = (p_vmem[0, pl.ds(i, 16)] * 4
                                      + c_vmem[0, pl.ds(i, 16)])

            iota64 = lax.iota(jnp.int32, 16) * S

            @pl.loop(0, S)
            def _(s):
                @pl.loop(0, BBLK, step=16)
                def _(j):
                    idx = iota64 + (j * S + s)
                    out_vmem[s, pl.ds(j, 16)] = plsc.load_gather(pc_v, [idx])

        pltpu.emit_pipeline(
            body,
            grid=(B // BBLK,),
            in_specs=[pl.BlockSpec((1, BBLK * S), lambda i: (0, i)),
                      pl.BlockSpec((1, BBLK * S), lambda i: (0, i))],
            out_specs=[pl.BlockSpec((S, BBLK), lambda i: (0, i))],
            core_axis_name=("core", "subcore"),
            dimension_semantics=(pltpu.PARALLEL,),
        )(p_hbm, c_hbm, out_hbm)

    return k(pieces_flat, colors_flat)


def _tc_body(tbl_ref, pc_ref, out_ref):
    tbl = tbl_ref[0]

    def col(k):
        return tbl[:, k][:, None]

    pk = pc_ref[0]
    b0 = (pk & 1) != 0
    b1 = (pk & 2) != 0
    b2 = (pk & 4) != 0
    b3 = (pk & 8) != 0
    b4 = (pk & 16) != 0
    t0 = jnp.where(b2, col(1), col(0))
    t1 = jnp.where(b2, col(3), col(2))
    t2 = jnp.where(b2, col(5), col(4))
    u0 = jnp.where(b3, t1, t0)
    u1 = jnp.where(b3, col(6), t2)
    pv = jnp.where(b4, u1, u0)
    cv = jnp.where(b1, col(10), jnp.where(b0, col(9), col(8)))
    out_ref[0] = pv + cv


def _tc_lookup(tbl, pcT):
    B = pcT.shape[2]
    return pl.pallas_call(
        _tc_body,
        grid=(S, B // BLKB),
        in_specs=[pl.BlockSpec((1, D, 16), lambda s, j: (s, 0, 0)),
                  pl.BlockSpec((1, 1, BLKB), lambda s, j: (s, 0, j))],
        out_specs=pl.BlockSpec((1, D, BLKB), lambda s, j: (s, 0, j)),
        out_shape=jax.ShapeDtypeStruct((S, D, B), jnp.float32),
    )(tbl, pcT)


def kernel(pieces_ids, color_ids, position_table, piece_table, color_table,
           indexes):
    B, _ = pieces_ids.shape
    pos = jnp.take(position_table, indexes, axis=0)
    pcols = piece_table[jnp.clip(jnp.arange(8), 0, 6)].T
    ccols = color_table[jnp.clip(jnp.arange(8), 0, 2)].T
    ptab = pos[:, :, None] + pcols[None, :, :]
    ctab = jnp.broadcast_to(ccols[None, :, :], (S, D, 8))
    tbl = jnp.concatenate([ptab, ctab], axis=-1)
    p = pieces_ids.reshape(1, B * S).astype(jnp.int32)
    c = color_ids.reshape(1, B * S).astype(jnp.int32)
    pcT = _sc_pc_transpose(p, c, B)
    out3 = _tc_lookup(tbl, pcT.reshape(S, 1, B))
    return jnp.transpose(out3, (2, 0, 1))


# speedup vs baseline: 24.5595x; 1.4339x over previous
import jax
import jax.numpy as jnp
from jax.experimental import pallas as pl

D = 64
S = 64
BLKB2 = 512


def _tc_body(tbl_ref, p_ref, c_ref, out_ref):
    import jax.lax as lax

    def chunk(si, carry):
        tbl = tbl_ref[si]

        def col(k, _tbl=tbl):
            return _tbl[:, k][:, None]

        pk = p_ref[si][None, :] * 4 + c_ref[si][None, :]
        b0 = (pk & 1) != 0
        b1 = (pk & 2) != 0
        b2 = (pk & 4) != 0
        b3 = (pk & 8) != 0
        b4 = (pk & 16) != 0
        t0 = jnp.where(b2, col(1), col(0))
        t1 = jnp.where(b2, col(3), col(2))
        t2 = jnp.where(b2, col(5), col(4))
        u0 = jnp.where(b3, t1, t0)
        u1 = jnp.where(b3, col(6), t2)
        pv = jnp.where(b4, u1, u0)
        cv = jnp.where(b1, col(10), jnp.where(b0, col(9), col(8)))
        out_ref[si] = pv + cv
        return carry

    lax.fori_loop(0, 8, chunk, 0, unroll=2)


def _tc_lookup(tbl, pT, cT):
    B = pT.shape[1]
    return pl.pallas_call(
        _tc_body,
        grid=(S // 8, B // BLKB2),
        in_specs=[pl.BlockSpec((8, D, 16), lambda s, j: (s, 0, 0)),
                  pl.BlockSpec((8, BLKB2), lambda s, j: (s, j)),
                  pl.BlockSpec((8, BLKB2), lambda s, j: (s, j))],
        out_specs=pl.BlockSpec((8, D, BLKB2), lambda s, j: (s, 0, j)),
        out_shape=jax.ShapeDtypeStruct((S, D, B), jnp.float32),
    )(tbl, pT, cT)


def kernel(pieces_ids, color_ids, position_table, piece_table, color_table,
           indexes):
    B, _ = pieces_ids.shape
    pos = jnp.take(position_table, indexes, axis=0)
    pcols = piece_table[jnp.clip(jnp.arange(8), 0, 6)].T
    ccols = color_table[jnp.clip(jnp.arange(8), 0, 2)].T
    ptab = pos[:, :, None] + pcols[None, :, :]
    ctab = jnp.broadcast_to(ccols[None, :, :], (S, D, 8))
    tbl = jnp.concatenate([ptab, ctab], axis=-1)
    pT = pieces_ids.astype(jnp.int32).T
    cT = color_ids.astype(jnp.int32).T
    out3 = _tc_lookup(tbl, pT, cT)
    return jnp.transpose(out3, (2, 0, 1))


# speedup vs baseline: 25.1374x; 1.0235x over previous
import jax
import jax.numpy as jnp
from jax.experimental import pallas as pl

D = 64
S = 64
BLKB2 = 512


def _tc_body(tbl_ref, p_ref, c_ref, out_ref):
    import jax.lax as lax

    def chunk(si, carry):
        tbl = tbl_ref[si]

        def col(k, _tbl=tbl):
            return _tbl[:, k][:, None]

        pk = p_ref[si][None, :] * 4 + c_ref[si][None, :]
        b0 = (pk & 1) != 0
        b1 = (pk & 2) != 0
        b2 = (pk & 4) != 0
        b3 = (pk & 8) != 0
        b4 = (pk & 16) != 0
        t0 = jnp.where(b2, col(1), col(0))
        t1 = jnp.where(b2, col(3), col(2))
        t2 = jnp.where(b2, col(5), col(4))
        u0 = jnp.where(b3, t1, t0)
        u1 = jnp.where(b3, col(6), t2)
        pv = jnp.where(b4, u1, u0)
        cv = jnp.where(b1, col(10), jnp.where(b0, col(9), col(8)))
        out_ref[si] = pv + cv
        return carry

    lax.fori_loop(0, 8, chunk, 0, unroll=4)


def _tc_lookup(tbl, pT, cT):
    B = pT.shape[1]
    return pl.pallas_call(
        _tc_body,
        grid=(S // 8, B // BLKB2),
        in_specs=[pl.BlockSpec((8, D, 16), lambda s, j: (s, 0, 0)),
                  pl.BlockSpec((8, BLKB2), lambda s, j: (s, j)),
                  pl.BlockSpec((8, BLKB2), lambda s, j: (s, j))],
        out_specs=pl.BlockSpec((8, D, BLKB2), lambda s, j: (s, 0, j)),
        out_shape=jax.ShapeDtypeStruct((S, D, B), jnp.float32),
    )(tbl, pT, cT)


def kernel(pieces_ids, color_ids, position_table, piece_table, color_table,
           indexes):
    B, _ = pieces_ids.shape
    pos = jnp.take(position_table, indexes, axis=0)
    pcols = piece_table[jnp.clip(jnp.arange(8), 0, 6)].T
    ccols = color_table[jnp.clip(jnp.arange(8), 0, 2)].T
    ptab = pos[:, :, None] + pcols[None, :, :]
    ctab = jnp.broadcast_to(ccols[None, :, :], (S, D, 8))
    tbl = jnp.concatenate([ptab, ctab], axis=-1)
    pT = pieces_ids.astype(jnp.int32).T
    cT = color_ids.astype(jnp.int32).T
    out3 = _tc_lookup(tbl, pT, cT)
    return jnp.transpose(out3, (2, 0, 1))


# speedup vs baseline: 42.1520x; 1.6769x over previous
import jax
import jax.numpy as jnp
from jax import lax
from jax.experimental import pallas as pl

D = 64
S = 64
BLKB2 = 2048


def _tc_body(tbl_ref, tbl21_ref, p_ref, c_ref, out_ref):
    def valu_square(si):
        tbl = tbl_ref[si]

        def col(k):
            return tbl[:, k][:, None]

        p = p_ref[si][None, :]
        c = c_ref[si][None, :]
        b0 = (c & 1) != 0
        b1 = (c & 2) != 0
        b2 = (p & 1) != 0
        b3 = (p & 2) != 0
        b4 = (p & 4) != 0
        t0 = jnp.where(b2, col(1), col(0))
        t1 = jnp.where(b2, col(3), col(2))
        t2 = jnp.where(b2, col(5), col(4))
        u0 = jnp.where(b3, t1, t0)
        u1 = jnp.where(b3, col(6), t2)
        pv = jnp.where(b4, u1, u0)
        cv = jnp.where(b1, col(10), jnp.where(b0, col(9), col(8)))
        out_ref[si] = pv + cv

    def xlu_square(si):
        tbl21 = tbl21_ref[si]
        pc = p_ref[si][None, :] * 3 + c_ref[si][None, :]
        idxb = jnp.broadcast_to(pc, (D, BLKB2))
        out_ref[si] = jnp.take_along_axis(tbl21, idxb, axis=1)

    def group(g, carry):
        base = g * 4
        xlu_square(base)
        valu_square(base + 1)
        valu_square(base + 2)
        valu_square(base + 3)
        return carry

    lax.fori_loop(0, 2, group, 0, unroll=2)


def _tc_lookup(tbl, tbl21, pT, cT):
    B = pT.shape[1]
    return pl.pallas_call(
        _tc_body,
        grid=(S // 8, B // BLKB2),
        in_specs=[pl.BlockSpec((8, D, 16), lambda s, j: (s, 0, 0)),
                  pl.BlockSpec((8, D, 32), lambda s, j: (s, 0, 0)),
                  pl.BlockSpec((8, BLKB2), lambda s, j: (s, j)),
                  pl.BlockSpec((8, BLKB2), lambda s, j: (s, j))],
        out_specs=pl.BlockSpec((8, D, BLKB2), lambda s, j: (s, 0, j)),
        out_shape=jax.ShapeDtypeStruct((S, D, B), jnp.float32),
    )(tbl, tbl21, pT, cT)


def kernel(pieces_ids, color_ids, position_table, piece_table, color_table,
           indexes):
    B, _ = pieces_ids.shape
    pos = jnp.take(position_table, indexes, axis=0)
    pcols = piece_table[jnp.clip(jnp.arange(8), 0, 6)].T
    ccols = color_table[jnp.clip(jnp.arange(8), 0, 2)].T
    ptab = pos[:, :, None] + pcols[None, :, :]
    ctab = jnp.broadcast_to(ccols[None, :, :], (S, D, 8))
    tbl = jnp.concatenate([ptab, ctab], axis=-1)
    pc32 = jnp.arange(32)
    p21 = piece_table[jnp.clip(pc32 // 3, 0, 6)].T
    c21 = color_table[pc32 % 3].T
    tbl21 = (pos[:, :, None] + p21[None, :, :]) + c21[None, :, :]
    pT = pieces_ids.astype(jnp.int32).T
    cT = color_ids.astype(jnp.int32).T
    out3 = _tc_lookup(tbl, tbl21, pT, cT)
    return jnp.transpose(out3, (2, 0, 1))
